# Initial kernel scaffold; baseline (speedup 1.0000x reference)
import jax
import jax.numpy as jnp
from jax.experimental import pallas as pl


def kernel(x, adj, W, b, is_sparse):
    raise NotImplementedError("write your pallas kernel here")


# speedup vs baseline: 1.0297x; 1.0297x over previous
import functools

import jax
import jax.numpy as jnp
from jax.experimental import pallas as pl
from jax.experimental.pallas import tpu as pltpu


def _gcn_stripe_kernel(x_ref, adj_ref, wt_ref, b_ref, out_ref, h_ref):
    @pl.when(pl.program_id(0) == 0)
    def _project():
        h = jnp.dot(x_ref[...], wt_ref[...], preferred_element_type=jnp.float32)
        h_ref[...] = h + b_ref[...]

    out_ref[...] = jnp.dot(adj_ref[...], h_ref[...],
                           preferred_element_type=jnp.float32)


@functools.partial(jax.jit, static_argnames=("bm",))
def _gcn(x, adj, wt, b, bm):
    n, d = x.shape
    grid = (n // bm,)
    return pl.pallas_call(
        _gcn_stripe_kernel,
        grid=grid,
        in_specs=[
            pl.BlockSpec((n, d), lambda i: (0, 0)),
            pl.BlockSpec((bm, n), lambda i: (i, 0)),
            pl.BlockSpec((d, d), lambda i: (0, 0)),
            pl.BlockSpec((1, d), lambda i: (0, 0)),
        ],
        out_specs=pl.BlockSpec((bm, d), lambda i: (i, 0)),
        out_shape=jax.ShapeDtypeStruct((n, d), jnp.float32),
        scratch_shapes=[pltpu.VMEM((n, d), jnp.float32)],
        compiler_params=pltpu.CompilerParams(
            dimension_semantics=("arbitrary",),
        ),
    )(x, adj, wt, b)


def kernel(x, adj, W, b, is_sparse):
    n, d = x.shape
    bm = 400 if n % 400 == 0 else n
    wt = W.T
    b2 = b.reshape(1, d)
    return _gcn(x, adj, wt, b2, bm)
